# baseline (speedup 1.0000x reference)
import functools

import jax
import jax.numpy as jnp
from jax import lax
from jax.experimental import pallas as pl
from jax.experimental.pallas import tpu as pltpu
from jax.experimental.pallas import tpu_sc as plsc

_CHUNK = 32
_NBUF = 2
_SBLK = 512


def kernel(token_ids, weight):
    batch_size, seq_len = token_ids.shape
    emb_dim = weight.shape[1]
    n_tc = batch_size - 1

    info = plsc.get_sparse_core_info()
    num_workers = info.num_cores * info.num_subcores
    rows_per = seq_len // num_workers
    nch = rows_per // _CHUNK

    mesh = plsc.VectorSubcoreMesh(core_axis_name="c", subcore_axis_name="s")

    @functools.partial(
        pl.kernel,
        mesh=mesh,
        out_type=jax.ShapeDtypeStruct((batch_size, seq_len, emb_dim), weight.dtype),
        scratch_types=[
            pltpu.VMEM((_NBUF, _CHUNK, emb_dim), weight.dtype),
            pltpu.SemaphoreType.DMA,
            pltpu.SemaphoreType.DMA,
        ],
    )
    def sc_copy(w_hbm, out_hbm, buf, gsem, ssem):
        wid = lax.axis_index("s") * info.num_cores + lax.axis_index("c")
        base = wid * rows_per

        gh = [None] * nch
        for i in range(_NBUF):
            gh[i] = pltpu.async_copy(
                w_hbm.at[pl.ds(base + i * _CHUNK, _CHUNK)], buf.at[i], gsem
            )
        for i in range(nch):
            gh[i].wait()
            scat = [
                pltpu.async_copy(
                    buf.at[i % _NBUF],
                    out_hbm.at[b, pl.ds(base + i * _CHUNK, _CHUNK)],
                    ssem,
                )
                for b in range(n_tc, batch_size)
            ]
            for s in scat:
                s.wait()
            nxt = i + _NBUF
            if nxt < nch:
                gh[nxt] = pltpu.async_copy(
                    w_hbm.at[pl.ds(base + nxt * _CHUNK, _CHUNK)],
                    buf.at[nxt % _NBUF],
                    gsem,
                )

    sc_out = sc_copy(weight)

    def tc_body(prev_ref, w_ref, o_ref):
        del prev_ref
        o_ref[...] = jnp.broadcast_to(w_ref[...][None], o_ref.shape)

    return pl.pallas_call(
        tc_body,
        grid=(seq_len // _SBLK,),
        in_specs=[
            pl.BlockSpec(memory_space=pl.ANY),
            pl.BlockSpec((_SBLK, emb_dim), lambda i: (i, 0)),
        ],
        out_specs=pl.BlockSpec((n_tc, _SBLK, emb_dim), lambda i: (0, i, 0)),
        out_shape=jax.ShapeDtypeStruct((batch_size, seq_len, emb_dim), weight.dtype),
        input_output_aliases={0: 0},
    )(sc_out, weight)


# speedup vs baseline: 1.1334x; 1.1334x over previous
import functools

import jax
import jax.numpy as jnp
from jax import lax
from jax.experimental import pallas as pl
from jax.experimental.pallas import tpu as pltpu
from jax.experimental.pallas import tpu_sc as plsc

_CHUNK = 32
_NBUF = 2


def kernel(token_ids, weight):
    batch_size, seq_len = token_ids.shape
    emb_dim = weight.shape[1]

    info = plsc.get_sparse_core_info()
    num_workers = info.num_cores * info.num_subcores
    rows_per = seq_len // num_workers
    nch = rows_per // _CHUNK

    mesh = plsc.VectorSubcoreMesh(core_axis_name="c", subcore_axis_name="s")

    @functools.partial(
        pl.kernel,
        mesh=mesh,
        out_type=jax.ShapeDtypeStruct((batch_size, seq_len, emb_dim), weight.dtype),
        scratch_types=[
            pltpu.VMEM((_NBUF, _CHUNK, emb_dim), weight.dtype),
            pltpu.SemaphoreType.DMA,
            pltpu.SemaphoreType.DMA,
        ],
    )
    def copy_kernel(w_hbm, out_hbm, buf, gsem, ssem):
        wid = lax.axis_index("s") * info.num_cores + lax.axis_index("c")
        base = wid * rows_per

        gh = [None] * nch
        for i in range(_NBUF):
            gh[i] = pltpu.async_copy(
                w_hbm.at[pl.ds(base + i * _CHUNK, _CHUNK)], buf.at[i], gsem
            )
        for i in range(nch):
            gh[i].wait()
            scat = [
                pltpu.async_copy(
                    buf.at[i % _NBUF],
                    out_hbm.at[b, pl.ds(base + i * _CHUNK, _CHUNK)],
                    ssem,
                )
                for b in range(batch_size)
            ]
            for s in scat:
                s.wait()
            nxt = i + _NBUF
            if nxt < nch:
                gh[nxt] = pltpu.async_copy(
                    w_hbm.at[pl.ds(base + nxt * _CHUNK, _CHUNK)],
                    buf.at[nxt % _NBUF],
                    gsem,
                )

    return copy_kernel(weight)
